# baseline (speedup 1.0000x reference)
import functools

import jax
import jax.numpy as jnp
from jax import lax
from jax.experimental import pallas as pl
from jax.experimental.pallas import tpu as pltpu
from jax.experimental.pallas import tpu_sc as plsc

_F32 = jnp.float32


def _take16(vec, idx):
    dnums = lax.GatherDimensionNumbers(offset_dims=(), collapsed_slice_dims=(0,),
                                       start_index_map=(0,))
    return lax.gather(vec, idx[:, None], dnums, (1,),
                      mode=lax.GatherScatterMode.PROMISE_IN_BOUNDS)


def _dot_t(a, w):
    return lax.dot_general(a, w, (((1,), (1,)), ((), ())),
                           preferred_element_type=_F32)


def _mm_t_body(x_ref, w_ref, o_ref):
    o_ref[...] = _dot_t(x_ref[...], w_ref[...])


def _mm_t_bias_body(a_ref, w_ref, b_ref, o_ref):
    y = _dot_t(a_ref[...], w_ref[...]) + b_ref[...]
    be, d = y.shape
    u = jax.lax.bitcast_convert_type(y, jnp.uint32)
    u = u + 0x7FFF + ((u >> 16) & 1)
    hb = u >> 16
    parts = [hb[:, c * 32:c * 32 + 16] | (hb[:, c * 32 + 16:c * 32 + 32] << 16)
             for c in range(d // 32)]
    o_ref[...] = jax.lax.bitcast_convert_type(
        jnp.concatenate(parts, axis=1), jnp.int32)


def _tc_xa(x, w):
    n, d = x.shape
    bn = 2000
    return pl.pallas_call(
        _mm_t_body,
        grid=(n // bn,),
        in_specs=[pl.BlockSpec((bn, d), lambda i: (i, 0)),
                  pl.BlockSpec((d, d), lambda i: (0, 0))],
        out_specs=pl.BlockSpec((bn, d), lambda i: (i, 0)),
        out_shape=jax.ShapeDtypeStruct((n, d), _F32),
    )(x, w)


def _tc_eaw(ea, w, b):
    e, de = ea.shape
    d = w.shape[0]
    be = 2000
    return pl.pallas_call(
        _mm_t_bias_body,
        grid=(e // be,),
        in_specs=[pl.BlockSpec((be, de), lambda i: (i, 0)),
                  pl.BlockSpec((d, de), lambda i: (0, 0)),
                  pl.BlockSpec((1, d), lambda i: (0, 0))],
        out_specs=pl.BlockSpec((be, d // 2), lambda i: (i, 0)),
        out_shape=jax.ShapeDtypeStruct((e, d // 2), jnp.int32),
    )(ea, w, b)


def _sc_edge_aggregate(xa, row, col, eaw):
    n, d = xa.shape
    e = row.shape[0]
    nc, ns = 2, 16
    ept = e // ns
    k = 80
    nch = ept // k
    npad = ((n + 2047) // 2048) * 2048
    nh = npad // nc
    rpt = nh // ns
    trows = nh + 8
    crows = nh // d
    chrows = crows + 8

    mesh = plsc.VectorSubcoreMesh(core_axis_name="c", subcore_axis_name="s",
                                  num_cores=nc, num_subcores=ns)

    @functools.partial(
        pl.kernel,
        out_type=(jax.ShapeDtypeStruct((npad, d), _F32),
                  jax.ShapeDtypeStruct((nc * crows, d), _F32)),
        mesh=mesh,
        compiler_params=pltpu.CompilerParams(needs_layout_passes=False),
        scratch_types=[
            pltpu.VMEM((k,), jnp.int32),
            pltpu.VMEM((k,), jnp.int32),
            pltpu.VMEM((k,), jnp.int32),
            pltpu.VMEM((k,), jnp.int32),
            pltpu.VMEM((k,), jnp.int32),
            pltpu.VMEM((k,), jnp.int32),
            pltpu.VMEM((k, d), _F32),
            pltpu.VMEM((k, d), _F32),
            pltpu.VMEM((k, d // 2), jnp.int32),
            pltpu.VMEM((k, d // 2), jnp.int32),
            pltpu.VMEM((k, d), _F32),
            pltpu.VMEM((k, d), _F32),
            pltpu.VMEM((rpt // 8, d), _F32),
            pltpu.VMEM((chrows, d), _F32),
            pltpu.VMEM((chrows,), jnp.int32),
            pltpu.VMEM_SHARED((trows, d), _F32),
            pltpu.VMEM_SHARED((chrows, d), _F32),
            pltpu.SemaphoreType.DMA,
            pltpu.SemaphoreType.DMA,
            pltpu.SemaphoreType.DMA,
            pltpu.SemaphoreType.DMA,
            pltpu.SemaphoreType.DMA,
            pltpu.SemaphoreType.DMA,
            pltpu.SemaphoreType.DMA,
            pltpu.SemaphoreType.DMA,
        ],
    )
    def body(xa_ref, row_ref, col_ref, eaw_ref, out_ref, cnt_ref,
             ri0, ri1, rc0, rc1, sc0, sc1, g0, g1, e0, e1, s0, s1,
             zbuf, hist, crid, acc, cacc,
             semg0, semg1, seme0, seme1, semsc0, semsc1, semi0, semi1):
        cid = lax.axis_index("c")
        sid = lax.axis_index("s")
        lo = cid * nh
        tile_base = sid * ept

        zero16 = jnp.zeros((16,), _F32)
        fones = jnp.ones((16,), _F32)
        lane = lax.iota(jnp.int32, 16)

        ri = (ri0, ri1)
        rc = (rc0, rc1)
        sc = (sc0, sc1)
        gb = (g0, g1)
        eb = (e0, e1)
        sb = (s0, s1)
        semg = (semg0, semg1)
        seme = (seme0, seme1)
        semsc = (semsc0, semsc1)
        semi = (semi0, semi1)

        def zrow(i, carry):
            for j in range(d // 16):
                zbuf[i, pl.ds(j * 16, 16)] = zero16
            return carry
        lax.fori_loop(0, rpt // 8, zrow, 0)

        def hrow(i, carry):
            for j in range(d // 16):
                hist[i, pl.ds(j * 16, 16)] = zero16
            return carry
        lax.fori_loop(0, chrows, hrow, 0)

        for g in range(chrows // 16):
            crid[pl.ds(g * 16, 16)] = lane + (g * 16)

        for b in range(8):
            pltpu.sync_copy(zbuf,
                            acc.at[pl.ds(sid * rpt + b * (rpt // 8), rpt // 8), :])

        @pl.when(sid == 0)
        def _():
            pltpu.sync_copy(zbuf.at[pl.ds(0, chrows), :], cacc)
        plsc.subcore_barrier()

        def issue_idx(p, t):
            base = tile_base + t * k
            pltpu.async_copy(row_ref.at[pl.ds(base, k)], ri[p], semi[p])
            pltpu.async_copy(col_ref.at[pl.ds(base, k)], rc[p], semi[p])

        def wait_idx(p):
            pltpu.make_async_copy(row_ref.at[pl.ds(0, k)], ri[p],
                                  semi[p]).wait()
            pltpu.make_async_copy(col_ref.at[pl.ds(0, k)], rc[p],
                                  semi[p]).wait()

        def issue_fetch(p, t):
            pltpu.async_copy(xa_ref.at[ri[p]], gb[p], semg[p])
            pltpu.async_copy(eaw_ref.at[pl.ds(tile_base + t * k, k), :],
                             eb[p], seme[p])

        issue_idx(0, 0)
        wait_idx(0)
        issue_fetch(0, 0)
        issue_idx(1, 1)

        def phase(t, t2, p):
            q = 1 - p
            tn = jnp.minimum(t + 1, nch - 1)
            wait_idx(q)
            issue_fetch(q, tn)
            pltpu.make_async_copy(xa_ref.at[ri[p]], gb[p], semg[p]).wait()
            pltpu.make_async_copy(
                eaw_ref.at[pl.ds(tile_base, k), :], eb[p], seme[p]).wait()

            @pl.when(t2 >= 1)
            def _():
                pltpu.make_async_copy(sb[p], acc.at[sc[p]], semsc[p]).wait()

            def rowbody(r4, c2):
                for u in range(4):
                    r = r4 * 4 + u
                    for c in range(d // 32):
                        ev = plsc.bitcast(eb[p][r, pl.ds(c * 16, 16)],
                                          jnp.bfloat16)
                        fe, fo = plsc.unpack(
                            ev, format=plsc.PackFormat.INTERLEAVED)
                        sl0 = pl.ds(c * 32, 16)
                        sl1 = pl.ds(c * 32 + 16, 16)
                        sb[p][r, sl0] = jnp.maximum(gb[p][r, sl0] + fe, 0.0)
                        sb[p][r, sl1] = jnp.maximum(gb[p][r, sl1] + fo, 0.0)
                return c2
            lax.fori_loop(0, k // 4, rowbody, 0)

            for g in range(k // 16):
                sl = pl.ds(g * 16, 16)
                v = rc[p][sl] - lo
                ok = jnp.logical_and(v >= 0, v < nh)
                sc[p][sl] = jnp.where(ok, v, nh)

            issue_idx(p, jnp.minimum(t + 2, nch - 1))
            pltpu.async_copy(sb[p], acc.at[sc[p]], semsc[p], add=True)

            def cgroup(g, c3):
                colv = sc[p][pl.ds(g * 16, 16)]
                r_i = jnp.right_shift(colv, 7)
                l_i = jnp.bitwise_and(colv, 127)
                for l in range(16):
                    plsc.addupdate_scatter(hist, [r_i, l_i], fones,
                                           mask=lane == l)
                return c3
            lax.fori_loop(0, k // 16, cgroup, jnp.int32(0))

        def outer(t2, carry):
            phase(2 * t2, t2, 0)
            phase(2 * t2 + 1, t2, 1)
            return carry
        lax.fori_loop(0, nch // 2, outer, 0)

        wait_idx(1)
        pltpu.make_async_copy(xa_ref.at[ri[0]], gb[0], semg[0]).wait()
        pltpu.make_async_copy(eaw_ref.at[pl.ds(tile_base, k), :],
                              eb[0], seme[0]).wait()
        pltpu.make_async_copy(sb[0], acc.at[sc[0]], semsc[0]).wait()
        pltpu.make_async_copy(sb[1], acc.at[sc[1]], semsc[1]).wait()

        pltpu.sync_copy(hist, cacc.at[crid], add=True)
        plsc.subcore_barrier()
        pltpu.sync_copy(acc.at[pl.ds(sid * rpt, rpt), :],
                        out_ref.at[pl.ds(cid * nh + sid * rpt, rpt), :])

        @pl.when(sid == 0)
        def _():
            pltpu.sync_copy(cacc.at[pl.ds(0, crows), :],
                            cnt_ref.at[pl.ds(cid * crows, crows), :])

    return body(xa, row, col, eaw)


def _node_body(p_ref, c4_ref, x_ref, b3_ref, u_ref, w1b_ref, b1b_ref, w2a_ref,
               b2a_ref, w2b_ref, b2b_ref, wv_ref, bv_ref, wo_ref, bo_ref,
               o_ref):
    d = x_ref.shape[1]
    bn = x_ref.shape[0]
    nb = u_ref.shape[0]
    sums = p_ref[...]
    cnt = jnp.reshape(c4_ref[...], (bn, 1))
    mean = sums / jnp.maximum(cnt, 1.0)
    agg = _dot_t(mean, w1b_ref[...]) + jnp.where(cnt > 0.0, 1.0, 0.0) * b1b_ref[...]
    c = _dot_t(u_ref[...], wv_ref[...]) + bv_ref[...]
    c = _dot_t(c, wo_ref[...]) + bo_ref[...]
    w2a = w2a_ref[...]
    cb = _dot_t(c, w2a[:, 2 * d:3 * d])
    bb = jnp.reshape(b3_ref[...], (bn, 1))
    onehot = (bb == lax.broadcasted_iota(jnp.int32, (bn, nb), 1)).astype(_F32)
    pre2 = (_dot_t(x_ref[...], w2a[:, :d])
            + _dot_t(agg, w2a[:, d:2 * d])
            + lax.dot_general(onehot, cb, (((1,), (0,)), ((), ())),
                              preferred_element_type=_F32)
            + b2a_ref[...])
    o_ref[...] = _dot_t(jnp.maximum(pre2, 0.0), w2b_ref[...]) + b2b_ref[...]


def _tc_node_mlp(p, c4, x, batch3, u, w1b, b1b, w2a, b2a, w2b, b2b, wv, bv,
                 wo, bo):
    n, d = x.shape
    nb = u.shape[0]
    bn = 1000
    grid = n // bn
    full = lambda *s: pl.BlockSpec(s, lambda i: tuple(0 for _ in s))
    return pl.pallas_call(
        _node_body,
        grid=(grid,),
        in_specs=[
            pl.BlockSpec((bn, d), lambda i: (i, 0)),
            pl.BlockSpec((1, 1, bn), lambda i: (i, 0, 0)),
            pl.BlockSpec((bn, d), lambda i: (i, 0)),
            pl.BlockSpec((1, 1, bn), lambda i: (i, 0, 0)),
            full(nb, d),
            full(d, d), full(1, d),
            full(d, 3 * d), full(1, d),
            full(d, d), full(1, d),
            full(d, d), full(1, d),
            full(d, d), full(1, d),
        ],
        out_specs=pl.BlockSpec((bn, d), lambda i: (i, 0)),
        out_shape=jax.ShapeDtypeStruct((n, d), _F32),
    )(p, c4, x, batch3, u, w1b, b1b, w2a, b2a, w2b, b2b, wv, bv, wo, bo)


def kernel(x, edge_index, edge_attr, u, batch, W1a, b1a, W1b, b1b, W2a, b2a,
           W2b, b2b, Wq, bq, Wk, bk, Wv, bv, Wo, bo):
    n, d = x.shape
    row = edge_index[0]
    col = edge_index[1]
    xa = _tc_xa(x, W1a[:, :d])
    eaw = _tc_eaw(edge_attr, W1a[:, d:], b1a.reshape(1, d))
    sums, cnts = _sc_edge_aggregate(xa, row, col, eaw)
    c4 = cnts.reshape(-1)[:n].reshape(n // 1000, 1, 1000)
    batch3 = batch.reshape(n // 1000, 1, 1000)
    return _tc_node_mlp(sums, c4, x, batch3, u,
                        W1b, b1b.reshape(1, d), W2a, b2a.reshape(1, d),
                        W2b, b2b.reshape(1, d), Wv, bv.reshape(1, d),
                        Wo, bo.reshape(1, d))


# speedup vs baseline: 1.1752x; 1.1752x over previous
import functools

import jax
import jax.numpy as jnp
from jax import lax
from jax.experimental import pallas as pl
from jax.experimental.pallas import tpu as pltpu
from jax.experimental.pallas import tpu_sc as plsc

_F32 = jnp.float32


def _take16(vec, idx):
    dnums = lax.GatherDimensionNumbers(offset_dims=(), collapsed_slice_dims=(0,),
                                       start_index_map=(0,))
    return lax.gather(vec, idx[:, None], dnums, (1,),
                      mode=lax.GatherScatterMode.PROMISE_IN_BOUNDS)


def _dot_t(a, w):
    return lax.dot_general(a, w, (((1,), (1,)), ((), ())),
                           preferred_element_type=_F32)


def _mm_t_body(x_ref, w_ref, o_ref):
    o_ref[...] = _dot_t(x_ref[...], w_ref[...])


def _mm_t_bias_body(a_ref, w_ref, b_ref, o_ref):
    o_ref[...] = _dot_t(a_ref[...], w_ref[...]) + b_ref[...]


def _tc_xa(x, w):
    n, d = x.shape
    bn = 2000
    return pl.pallas_call(
        _mm_t_body,
        grid=(n // bn,),
        in_specs=[pl.BlockSpec((bn, d), lambda i: (i, 0)),
                  pl.BlockSpec((d, d), lambda i: (0, 0))],
        out_specs=pl.BlockSpec((bn, d), lambda i: (i, 0)),
        out_shape=jax.ShapeDtypeStruct((n, d), _F32),
    )(x, w)


def _tc_eaw(ea, w, b, blk0, nblk):
    e, de = ea.shape
    d = w.shape[0]
    be = 2000
    return pl.pallas_call(
        _mm_t_bias_body,
        grid=(nblk,),
        in_specs=[pl.BlockSpec((be, de), lambda i: (i + blk0, 0)),
                  pl.BlockSpec((d, de), lambda i: (0, 0)),
                  pl.BlockSpec((1, d), lambda i: (0, 0))],
        out_specs=pl.BlockSpec((be, d), lambda i: (i, 0)),
        out_shape=jax.ShapeDtypeStruct((nblk * be, d), _F32),
    )(ea, w, b)


def _sc_edge_aggregate(xa, row, col, eaw, eoff):
    n, d = xa.shape
    e = eaw.shape[0]
    nc, ns = 2, 16
    ept = e // ns
    k = 80
    nch = ept // k
    npad = ((n + 2047) // 2048) * 2048
    nh = npad // nc
    rpt = nh // ns
    trows = nh + 8
    crows = nh // d
    chrows = crows + 8

    mesh = plsc.VectorSubcoreMesh(core_axis_name="c", subcore_axis_name="s",
                                  num_cores=nc, num_subcores=ns)

    @functools.partial(
        pl.kernel,
        out_type=(jax.ShapeDtypeStruct((npad, d), _F32),
                  jax.ShapeDtypeStruct((nc * crows, d), _F32)),
        mesh=mesh,
        compiler_params=pltpu.CompilerParams(needs_layout_passes=False),
        scratch_types=[
            pltpu.VMEM((k,), jnp.int32),
            pltpu.VMEM((k,), jnp.int32),
            pltpu.VMEM((k,), jnp.int32),
            pltpu.VMEM((k,), jnp.int32),
            pltpu.VMEM((k,), jnp.int32),
            pltpu.VMEM((k,), jnp.int32),
            pltpu.VMEM((k, d), _F32),
            pltpu.VMEM((k, d), _F32),
            pltpu.VMEM((k, d), _F32),
            pltpu.VMEM((k, d), _F32),
            pltpu.VMEM((k, d), _F32),
            pltpu.VMEM((k, d), _F32),
            pltpu.VMEM((rpt // 8, d), _F32),
            pltpu.VMEM((chrows, d), _F32),
            pltpu.VMEM((chrows,), jnp.int32),
            pltpu.VMEM_SHARED((trows, d), _F32),
            pltpu.VMEM_SHARED((chrows, d), _F32),
            pltpu.SemaphoreType.DMA,
            pltpu.SemaphoreType.DMA,
            pltpu.SemaphoreType.DMA,
            pltpu.SemaphoreType.DMA,
            pltpu.SemaphoreType.DMA,
            pltpu.SemaphoreType.DMA,
            pltpu.SemaphoreType.DMA,
            pltpu.SemaphoreType.DMA,
        ],
    )
    def body(xa_ref, row_ref, col_ref, eaw_ref, out_ref, cnt_ref,
             ri0, ri1, rc0, rc1, sc0, sc1, g0, g1, e0, e1, s0, s1,
             zbuf, hist, crid, acc, cacc,
             semg0, semg1, seme0, seme1, semsc0, semsc1, semi0, semi1):
        cid = lax.axis_index("c")
        sid = lax.axis_index("s")
        lo = cid * nh
        tile_base = sid * ept
        idx_base = eoff + tile_base

        zero16 = jnp.zeros((16,), _F32)
        fones = jnp.ones((16,), _F32)
        lane = lax.iota(jnp.int32, 16)

        ri = (ri0, ri1)
        rc = (rc0, rc1)
        sc = (sc0, sc1)
        gb = (g0, g1)
        eb = (e0, e1)
        sb = (s0, s1)
        semg = (semg0, semg1)
        seme = (seme0, seme1)
        semsc = (semsc0, semsc1)
        semi = (semi0, semi1)

        def zrow(i, carry):
            for j in range(d // 16):
                zbuf[i, pl.ds(j * 16, 16)] = zero16
            return carry
        lax.fori_loop(0, rpt // 8, zrow, 0)

        def hrow(i, carry):
            for j in range(d // 16):
                hist[i, pl.ds(j * 16, 16)] = zero16
            return carry
        lax.fori_loop(0, chrows, hrow, 0)

        for g in range(chrows // 16):
            crid[pl.ds(g * 16, 16)] = lane + (g * 16)

        for b in range(8):
            pltpu.sync_copy(zbuf,
                            acc.at[pl.ds(sid * rpt + b * (rpt // 8), rpt // 8), :])

        @pl.when(sid == 0)
        def _():
            pltpu.sync_copy(zbuf.at[pl.ds(0, chrows), :], cacc)
        plsc.subcore_barrier()

        def issue_idx(p, t):
            base = idx_base + t * k
            pltpu.async_copy(row_ref.at[pl.ds(base, k)], ri[p], semi[p])
            pltpu.async_copy(col_ref.at[pl.ds(base, k)], rc[p], semi[p])

        def wait_idx(p):
            pltpu.make_async_copy(row_ref.at[pl.ds(0, k)], ri[p],
                                  semi[p]).wait()
            pltpu.make_async_copy(col_ref.at[pl.ds(0, k)], rc[p],
                                  semi[p]).wait()

        def issue_fetch(p, t):
            pltpu.async_copy(xa_ref.at[ri[p]], gb[p], semg[p])
            pltpu.async_copy(eaw_ref.at[pl.ds(tile_base + t * k, k), :],
                             eb[p], seme[p])

        issue_idx(0, 0)
        wait_idx(0)
        issue_fetch(0, 0)
        issue_idx(1, 1)

        def phase(t, t2, p):
            q = 1 - p
            tn = jnp.minimum(t + 1, nch - 1)
            wait_idx(q)
            issue_fetch(q, tn)
            pltpu.make_async_copy(xa_ref.at[ri[p]], gb[p], semg[p]).wait()
            pltpu.make_async_copy(
                eaw_ref.at[pl.ds(tile_base, k), :], eb[p], seme[p]).wait()

            @pl.when(t2 >= 1)
            def _():
                pltpu.make_async_copy(sb[p], acc.at[sc[p]], semsc[p]).wait()

            def rowbody(r4, c2):
                for u in range(4):
                    r = r4 * 4 + u
                    for j in range(d // 16):
                        sl = pl.ds(j * 16, 16)
                        sb[p][r, sl] = jnp.maximum(
                            gb[p][r, sl] + eb[p][r, sl], 0.0)
                return c2
            lax.fori_loop(0, k // 4, rowbody, 0)

            for g in range(k // 16):
                sl = pl.ds(g * 16, 16)
                v = rc[p][sl] - lo
                ok = jnp.logical_and(v >= 0, v < nh)
                sc[p][sl] = jnp.where(ok, v, nh)

            issue_idx(p, jnp.minimum(t + 2, nch - 1))
            pltpu.async_copy(sb[p], acc.at[sc[p]], semsc[p], add=True)

            def cgroup(g, c3):
                colv = sc[p][pl.ds(g * 16, 16)]
                r_i = jnp.right_shift(colv, 7)
                l_i = jnp.bitwise_and(colv, 127)
                for l in range(16):
                    plsc.addupdate_scatter(hist, [r_i, l_i], fones,
                                           mask=lane == l)
                return c3
            lax.fori_loop(0, k // 16, cgroup, jnp.int32(0))

        def outer(t2, carry):
            phase(2 * t2, t2, 0)
            phase(2 * t2 + 1, t2, 1)
            return carry
        lax.fori_loop(0, nch // 2, outer, 0)
        if nch % 2 == 1:
            phase(nch - 1, nch // 2, (nch - 1) % 2)

        pl_last = (nch - 1) % 2
        q_last = 1 - pl_last
        wait_idx(pl_last)
        pltpu.make_async_copy(xa_ref.at[ri[q_last]], gb[q_last],
                              semg[q_last]).wait()
        pltpu.make_async_copy(eaw_ref.at[pl.ds(tile_base, k), :],
                              eb[q_last], seme[q_last]).wait()
        pltpu.make_async_copy(sb[0], acc.at[sc[0]], semsc[0]).wait()
        pltpu.make_async_copy(sb[1], acc.at[sc[1]], semsc[1]).wait()

        pltpu.sync_copy(hist, cacc.at[crid], add=True)
        plsc.subcore_barrier()
        pltpu.sync_copy(acc.at[pl.ds(sid * rpt, rpt), :],
                        out_ref.at[pl.ds(cid * nh + sid * rpt, rpt), :])

        @pl.when(sid == 0)
        def _():
            pltpu.sync_copy(cacc.at[pl.ds(0, crows), :],
                            cnt_ref.at[pl.ds(cid * crows, crows), :])

    return body(xa, row, col, eaw)


def _node_body(p_ref, p2_ref, c4_ref, c42_ref, x_ref, b3_ref, u_ref, w1b_ref,
               b1b_ref, w2a_ref, b2a_ref, w2b_ref, b2b_ref, wv_ref, bv_ref,
               wo_ref, bo_ref, o_ref):
    d = x_ref.shape[1]
    bn = x_ref.shape[0]
    nb = u_ref.shape[0]
    sums = p_ref[...] + p2_ref[...]
    cnt = jnp.reshape(c4_ref[...] + c42_ref[...], (bn, 1))
    mean = sums / jnp.maximum(cnt, 1.0)
    agg = _dot_t(mean, w1b_ref[...]) + jnp.where(cnt > 0.0, 1.0, 0.0) * b1b_ref[...]
    c = _dot_t(u_ref[...], wv_ref[...]) + bv_ref[...]
    c = _dot_t(c, wo_ref[...]) + bo_ref[...]
    w2a = w2a_ref[...]
    cb = _dot_t(c, w2a[:, 2 * d:3 * d])
    bb = jnp.reshape(b3_ref[...], (bn, 1))
    onehot = (bb == lax.broadcasted_iota(jnp.int32, (bn, nb), 1)).astype(_F32)
    pre2 = (_dot_t(x_ref[...], w2a[:, :d])
            + _dot_t(agg, w2a[:, d:2 * d])
            + lax.dot_general(onehot, cb, (((1,), (0,)), ((), ())),
                              preferred_element_type=_F32)
            + b2a_ref[...])
    o_ref[...] = _dot_t(jnp.maximum(pre2, 0.0), w2b_ref[...]) + b2b_ref[...]


def _tc_node_mlp(p, p2, c4, c42, x, batch3, u, w1b, b1b, w2a, b2a, w2b, b2b,
                 wv, bv, wo, bo):
    n, d = x.shape
    nb = u.shape[0]
    bn = 1000
    grid = n // bn
    full = lambda *s: pl.BlockSpec(s, lambda i: tuple(0 for _ in s))
    return pl.pallas_call(
        _node_body,
        grid=(grid,),
        in_specs=[
            pl.BlockSpec((bn, d), lambda i: (i, 0)),
            pl.BlockSpec((bn, d), lambda i: (i, 0)),
            pl.BlockSpec((1, 1, bn), lambda i: (i, 0, 0)),
            pl.BlockSpec((1, 1, bn), lambda i: (i, 0, 0)),
            pl.BlockSpec((bn, d), lambda i: (i, 0)),
            pl.BlockSpec((1, 1, bn), lambda i: (i, 0, 0)),
            full(nb, d),
            full(d, d), full(1, d),
            full(d, 3 * d), full(1, d),
            full(d, d), full(1, d),
            full(d, d), full(1, d),
            full(d, d), full(1, d),
        ],
        out_specs=pl.BlockSpec((bn, d), lambda i: (i, 0)),
        out_shape=jax.ShapeDtypeStruct((n, d), _F32),
    )(p, p2, c4, c42, x, batch3, u, w1b, b1b, w2a, b2a, w2b, b2b, wv, bv,
      wo, bo)


def kernel(x, edge_index, edge_attr, u, batch, W1a, b1a, W1b, b1b, W2a, b2a,
           W2b, b2b, Wq, bq, Wk, bk, Wv, bv, Wo, bo):
    n, d = x.shape
    row = edge_index[0]
    col = edge_index[1]
    e = edge_attr.shape[0]
    be = 2000
    nblk_half = e // (2 * be)
    xa = _tc_xa(x, W1a[:, :d])
    w1ae = W1a[:, d:]
    b1a2 = b1a.reshape(1, d)
    eaw_a = _tc_eaw(edge_attr, w1ae, b1a2, 0, nblk_half)
    sums_a, cnts_a = _sc_edge_aggregate(xa, row, col, eaw_a, 0)
    eaw_b = _tc_eaw(edge_attr, w1ae, b1a2, nblk_half, nblk_half)
    sums_b, cnts_b = _sc_edge_aggregate(xa, row, col, eaw_b, e // 2)
    c4a = cnts_a.reshape(-1)[:n].reshape(n // 1000, 1, 1000)
    c4b = cnts_b.reshape(-1)[:n].reshape(n // 1000, 1, 1000)
    batch3 = batch.reshape(n // 1000, 1, 1000)
    return _tc_node_mlp(sums_a, sums_b, c4a, c4b, x, batch3, u,
                        W1b, b1b.reshape(1, d), W2a, b2a.reshape(1, d),
                        W2b, b2b.reshape(1, d), Wv, bv.reshape(1, d),
                        Wo, bo.reshape(1, d))


# speedup vs baseline: 1.1805x; 1.0045x over previous
import functools

import jax
import jax.numpy as jnp
from jax import lax
from jax.experimental import pallas as pl
from jax.experimental.pallas import tpu as pltpu
from jax.experimental.pallas import tpu_sc as plsc

_F32 = jnp.float32


def _take16(vec, idx):
    dnums = lax.GatherDimensionNumbers(offset_dims=(), collapsed_slice_dims=(0,),
                                       start_index_map=(0,))
    return lax.gather(vec, idx[:, None], dnums, (1,),
                      mode=lax.GatherScatterMode.PROMISE_IN_BOUNDS)


def _dot_t(a, w):
    return lax.dot_general(a, w, (((1,), (1,)), ((), ())),
                           preferred_element_type=_F32)


def _rnd_bf16_bits(y):
    u = jax.lax.bitcast_convert_type(y, jnp.uint32)
    return (u + 0x7FFF + ((u >> 16) & 1)) >> 16


def _mm_t_pack_body(a_ref, w_ref, b_ref, o_ref):
    y = _dot_t(a_ref[...], w_ref[...]) + b_ref[...]
    h = y.shape[1] // 2
    lo = _rnd_bf16_bits(y[:, :h])
    hi = _rnd_bf16_bits(y[:, h:])
    o_ref[...] = jax.lax.bitcast_convert_type(lo | (hi << 16), jnp.int32)


def _mm_t_body(x_ref, w_ref, o_ref):
    o_ref[...] = _dot_t(x_ref[...], w_ref[...])


def _tc_xa(x, w):
    n, d = x.shape
    bn = 2000
    return pl.pallas_call(
        _mm_t_body,
        grid=(n // bn,),
        in_specs=[pl.BlockSpec((bn, d), lambda i: (i, 0)),
                  pl.BlockSpec((d, d), lambda i: (0, 0))],
        out_specs=pl.BlockSpec((bn, d), lambda i: (i, 0)),
        out_shape=jax.ShapeDtypeStruct((n, d), _F32),
    )(x, w)


def _tc_pack2(a, w, b, blk0, nblk):
    de = a.shape[1]
    d = w.shape[0]
    be = 2000
    return pl.pallas_call(
        _mm_t_pack_body,
        grid=(nblk,),
        in_specs=[pl.BlockSpec((be, de), lambda i: (i + blk0, 0)),
                  pl.BlockSpec((d, de), lambda i: (0, 0)),
                  pl.BlockSpec((1, d), lambda i: (0, 0))],
        out_specs=pl.BlockSpec((be, d // 2), lambda i: (i, 0)),
        out_shape=jax.ShapeDtypeStruct((nblk * be, d // 2), jnp.int32),
    )(a, w, b)


def _sc_edge_aggregate(xa, row, col, eaw, eoff):
    n, d = xa.shape
    e = eaw.shape[0]
    nc, ns = 2, 16
    ept = e // ns
    k = 80
    nch = ept // k
    npad = ((n + 2047) // 2048) * 2048
    nh = npad // nc
    rpt = nh // ns
    trows = nh + 8
    crows = nh // d
    chrows = crows + 8

    mesh = plsc.VectorSubcoreMesh(core_axis_name="c", subcore_axis_name="s",
                                  num_cores=nc, num_subcores=ns)

    @functools.partial(
        pl.kernel,
        out_type=(jax.ShapeDtypeStruct((npad, d), _F32),
                  jax.ShapeDtypeStruct((nc * crows, d), _F32)),
        mesh=mesh,
        compiler_params=pltpu.CompilerParams(needs_layout_passes=False),
        scratch_types=[
            pltpu.VMEM((k,), jnp.int32),
            pltpu.VMEM((k,), jnp.int32),
            pltpu.VMEM((k,), jnp.int32),
            pltpu.VMEM((k,), jnp.int32),
            pltpu.VMEM((k,), jnp.int32),
            pltpu.VMEM((k,), jnp.int32),
            pltpu.VMEM((k, d), _F32),
            pltpu.VMEM((k, d), _F32),
            pltpu.VMEM((k, d // 2), jnp.int32),
            pltpu.VMEM((k, d // 2), jnp.int32),
            pltpu.VMEM((k, d), _F32),
            pltpu.VMEM((k, d), _F32),
            pltpu.VMEM((rpt // 8, d), _F32),
            pltpu.VMEM((chrows, d), _F32),
            pltpu.VMEM((chrows,), jnp.int32),
            pltpu.VMEM_SHARED((trows, d), _F32),
            pltpu.VMEM_SHARED((chrows, d), _F32),
            pltpu.SemaphoreType.DMA,
            pltpu.SemaphoreType.DMA,
            pltpu.SemaphoreType.DMA,
            pltpu.SemaphoreType.DMA,
            pltpu.SemaphoreType.DMA,
            pltpu.SemaphoreType.DMA,
            pltpu.SemaphoreType.DMA,
            pltpu.SemaphoreType.DMA,
        ],
    )
    def body(xa_ref, row_ref, col_ref, eaw_ref, out_ref, cnt_ref,
             ri0, ri1, rc0, rc1, sc0, sc1, g0, g1, e0, e1, s0, s1,
             zbuf, hist, crid, acc, cacc,
             semg0, semg1, seme0, seme1, semsc0, semsc1, semi0, semi1):
        cid = lax.axis_index("c")
        sid = lax.axis_index("s")
        lo = cid * nh
        tile_base = sid * ept
        idx_base = eoff + tile_base

        zero16 = jnp.zeros((16,), _F32)
        fones = jnp.ones((16,), _F32)
        lane = lax.iota(jnp.int32, 16)

        ri = (ri0, ri1)
        rc = (rc0, rc1)
        sc = (sc0, sc1)
        gb = (g0, g1)
        eb = (e0, e1)
        sb = (s0, s1)
        semg = (semg0, semg1)
        seme = (seme0, seme1)
        semsc = (semsc0, semsc1)
        semi = (semi0, semi1)

        def zrow(i, carry):
            for j in range(d // 16):
                zbuf[i, pl.ds(j * 16, 16)] = zero16
            return carry
        lax.fori_loop(0, rpt // 8, zrow, 0)

        def hrow(i, carry):
            for j in range(d // 16):
                hist[i, pl.ds(j * 16, 16)] = zero16
            return carry
        lax.fori_loop(0, chrows, hrow, 0)

        for g in range(chrows // 16):
            crid[pl.ds(g * 16, 16)] = lane + (g * 16)

        for b in range(8):
            pltpu.sync_copy(zbuf,
                            acc.at[pl.ds(sid * rpt + b * (rpt // 8), rpt // 8), :])

        @pl.when(sid == 0)
        def _():
            pltpu.sync_copy(zbuf.at[pl.ds(0, chrows), :], cacc)
        plsc.subcore_barrier()

        def issue_idx(p, t):
            base = idx_base + t * k
            pltpu.async_copy(row_ref.at[pl.ds(base, k)], ri[p], semi[p])
            pltpu.async_copy(col_ref.at[pl.ds(base, k)], rc[p], semi[p])

        def wait_idx(p):
            pltpu.make_async_copy(row_ref.at[pl.ds(0, k)], ri[p],
                                  semi[p]).wait()
            pltpu.make_async_copy(col_ref.at[pl.ds(0, k)], rc[p],
                                  semi[p]).wait()

        def issue_fetch(p, t):
            pltpu.async_copy(xa_ref.at[ri[p]], gb[p], semg[p])
            pltpu.async_copy(eaw_ref.at[pl.ds(tile_base + t * k, k), :],
                             eb[p], seme[p])

        issue_idx(0, 0)
        wait_idx(0)
        issue_fetch(0, 0)
        issue_idx(1, 1)

        def phase(t, t2, p):
            q = 1 - p
            tn = jnp.minimum(t + 1, nch - 1)
            wait_idx(q)
            issue_fetch(q, tn)
            pltpu.make_async_copy(xa_ref.at[ri[p]], gb[p], semg[p]).wait()
            pltpu.make_async_copy(
                eaw_ref.at[pl.ds(tile_base, k), :], eb[p], seme[p]).wait()

            @pl.when(t2 >= 1)
            def _():
                pltpu.make_async_copy(sb[p], acc.at[sc[p]], semsc[p]).wait()

            def rowbody(r4, c2):
                for u in range(4):
                    r = r4 * 4 + u
                    for c in range(d // 32):
                        ev = plsc.bitcast(eb[p][r, pl.ds(c * 16, 16)],
                                          jnp.bfloat16)
                        fe, fo = plsc.unpack(
                            ev, format=plsc.PackFormat.INTERLEAVED)
                        sl_lo = pl.ds(c * 16, 16)
                        sl_hi = pl.ds(d // 2 + c * 16, 16)
                        sb[p][r, sl_lo] = jnp.maximum(gb[p][r, sl_lo] + fe,
                                                      0.0)
                        sb[p][r, sl_hi] = jnp.maximum(gb[p][r, sl_hi] + fo,
                                                      0.0)
                return c2
            lax.fori_loop(0, k // 4, rowbody, 0)

            for g in range(k // 16):
                sl = pl.ds(g * 16, 16)
                v = rc[p][sl] - lo
                ok = jnp.logical_and(v >= 0, v < nh)
                sc[p][sl] = jnp.where(ok, v, nh)

            issue_idx(p, jnp.minimum(t + 2, nch - 1))
            pltpu.async_copy(sb[p], acc.at[sc[p]], semsc[p], add=True)

            def cgroup(g, c3):
                colv = sc[p][pl.ds(g * 16, 16)]
                r_i = jnp.right_shift(colv, 7)
                l_i = jnp.bitwise_and(colv, 127)
                for l in range(16):
                    plsc.addupdate_scatter(hist, [r_i, l_i], fones,
                                           mask=lane == l)
                return c3
            lax.fori_loop(0, k // 16, cgroup, jnp.int32(0))

        def outer(t2, carry):
            phase(2 * t2, t2, 0)
            phase(2 * t2 + 1, t2, 1)
            return carry
        lax.fori_loop(0, nch // 2, outer, 0)
        if nch % 2 == 1:
            phase(nch - 1, nch // 2, (nch - 1) % 2)

        pl_last = (nch - 1) % 2
        q_last = 1 - pl_last
        wait_idx(pl_last)
        pltpu.make_async_copy(xa_ref.at[ri[q_last]], gb[q_last],
                              semg[q_last]).wait()
        pltpu.make_async_copy(eaw_ref.at[pl.ds(tile_base, k), :],
                              eb[q_last], seme[q_last]).wait()
        pltpu.make_async_copy(sb[0], acc.at[sc[0]], semsc[0]).wait()
        pltpu.make_async_copy(sb[1], acc.at[sc[1]], semsc[1]).wait()

        pltpu.sync_copy(hist, cacc.at[crid], add=True)
        plsc.subcore_barrier()
        pltpu.sync_copy(acc.at[pl.ds(sid * rpt, rpt), :],
                        out_ref.at[pl.ds(cid * nh + sid * rpt, rpt), :])

        @pl.when(sid == 0)
        def _():
            pltpu.sync_copy(cacc.at[pl.ds(0, crows), :],
                            cnt_ref.at[pl.ds(cid * crows, crows), :])

    return body(xa, row, col, eaw)


def _node_body(p_ref, p2_ref, c4_ref, c42_ref, x_ref, b3_ref, u_ref, w1b_ref,
               b1b_ref, w2a_ref, b2a_ref, w2b_ref, b2b_ref, wv_ref, bv_ref,
               wo_ref, bo_ref, o_ref):
    d = x_ref.shape[1]
    bn = x_ref.shape[0]
    nb = u_ref.shape[0]
    sums = p_ref[...] + p2_ref[...]
    cnt = jnp.reshape(c4_ref[...] + c42_ref[...], (bn, 1))
    mean = sums / jnp.maximum(cnt, 1.0)
    agg = _dot_t(mean, w1b_ref[...]) + jnp.where(cnt > 0.0, 1.0, 0.0) * b1b_ref[...]
    c = _dot_t(u_ref[...], wv_ref[...]) + bv_ref[...]
    c = _dot_t(c, wo_ref[...]) + bo_ref[...]
    w2a = w2a_ref[...]
    cb = _dot_t(c, w2a[:, 2 * d:3 * d])
    bb = jnp.reshape(b3_ref[...], (bn, 1))
    onehot = (bb == lax.broadcasted_iota(jnp.int32, (bn, nb), 1)).astype(_F32)
    pre2 = (_dot_t(x_ref[...], w2a[:, :d])
            + _dot_t(agg, w2a[:, d:2 * d])
            + lax.dot_general(onehot, cb, (((1,), (0,)), ((), ())),
                              preferred_element_type=_F32)
            + b2a_ref[...])
    o_ref[...] = _dot_t(jnp.maximum(pre2, 0.0), w2b_ref[...]) + b2b_ref[...]


def _tc_node_mlp(p, p2, c4, c42, x, batch3, u, w1b, b1b, w2a, b2a, w2b, b2b,
                 wv, bv, wo, bo):
    n, d = x.shape
    nb = u.shape[0]
    bn = 1000
    grid = n // bn
    full = lambda *s: pl.BlockSpec(s, lambda i: tuple(0 for _ in s))
    return pl.pallas_call(
        _node_body,
        grid=(grid,),
        in_specs=[
            pl.BlockSpec((bn, d), lambda i: (i, 0)),
            pl.BlockSpec((bn, d), lambda i: (i, 0)),
            pl.BlockSpec((1, 1, bn), lambda i: (i, 0, 0)),
            pl.BlockSpec((1, 1, bn), lambda i: (i, 0, 0)),
            pl.BlockSpec((bn, d), lambda i: (i, 0)),
            pl.BlockSpec((1, 1, bn), lambda i: (i, 0, 0)),
            full(nb, d),
            full(d, d), full(1, d),
            full(d, 3 * d), full(1, d),
            full(d, d), full(1, d),
            full(d, d), full(1, d),
            full(d, d), full(1, d),
        ],
        out_specs=pl.BlockSpec((bn, d), lambda i: (i, 0)),
        out_shape=jax.ShapeDtypeStruct((n, d), _F32),
    )(p, p2, c4, c42, x, batch3, u, w1b, b1b, w2a, b2a, w2b, b2b, wv, bv,
      wo, bo)


def kernel(x, edge_index, edge_attr, u, batch, W1a, b1a, W1b, b1b, W2a, b2a,
           W2b, b2b, Wq, bq, Wk, bk, Wv, bv, Wo, bo):
    n, d = x.shape
    row = edge_index[0]
    col = edge_index[1]
    e = edge_attr.shape[0]
    be = 2000
    nblk_half = e // (2 * be)
    xa = _tc_xa(x, W1a[:, :d])
    w1ae = W1a[:, d:]
    b1a2 = b1a.reshape(1, d)
    eaw_a = _tc_pack2(edge_attr, w1ae, b1a2, 0, nblk_half)
    sums_a, cnts_a = _sc_edge_aggregate(xa, row, col, eaw_a, 0)
    eaw_b = _tc_pack2(edge_attr, w1ae, b1a2, nblk_half, nblk_half)
    sums_b, cnts_b = _sc_edge_aggregate(xa, row, col, eaw_b, e // 2)
    c4a = cnts_a.reshape(-1)[:n].reshape(n // 1000, 1, 1000)
    c4b = cnts_b.reshape(-1)[:n].reshape(n // 1000, 1, 1000)
    batch3 = batch.reshape(n // 1000, 1, 1000)
    return _tc_node_mlp(sums_a, sums_b, c4a, c4b, x, batch3, u,
                        W1b, b1b.reshape(1, d), W2a, b2a.reshape(1, d),
                        W2b, b2b.reshape(1, d), Wv, bv.reshape(1, d),
                        Wo, bo.reshape(1, d))


# speedup vs baseline: 1.1811x; 1.0005x over previous
import functools

import jax
import jax.numpy as jnp
from jax import lax
from jax.experimental import pallas as pl
from jax.experimental.pallas import tpu as pltpu
from jax.experimental.pallas import tpu_sc as plsc

_F32 = jnp.float32


def _dot_t(a, w):
    return lax.dot_general(a, w, (((1,), (1,)), ((), ())),
                           preferred_element_type=_F32)


def _rnd_bf16_bits(y):
    u = jax.lax.bitcast_convert_type(y, jnp.uint32)
    return (u + 0x7FFF + ((u >> 16) & 1)) >> 16


def _mm_t_pack_body(a_ref, w_ref, b_ref, o_ref):
    y = _dot_t(a_ref[...], w_ref[...]) + b_ref[...]
    h = y.shape[1] // 2
    lo = _rnd_bf16_bits(y[:, :h])
    hi = _rnd_bf16_bits(y[:, h:])
    o_ref[...] = jax.lax.bitcast_convert_type(lo | (hi << 16), jnp.int32)


def _mm_t_body(x_ref, w_ref, o_ref):
    o_ref[...] = _dot_t(x_ref[...], w_ref[...])


def _tc_xa(x, w):
    n, d = x.shape
    bn = 2000
    return pl.pallas_call(
        _mm_t_body,
        grid=(n // bn,),
        in_specs=[pl.BlockSpec((bn, d), lambda i: (i, 0)),
                  pl.BlockSpec((d, d), lambda i: (0, 0))],
        out_specs=pl.BlockSpec((bn, d), lambda i: (i, 0)),
        out_shape=jax.ShapeDtypeStruct((n, d), _F32),
    )(x, w)


def _tc_pack2(a, w, b, blk0, nblk):
    de = a.shape[1]
    d = w.shape[0]
    be = 2000
    return pl.pallas_call(
        _mm_t_pack_body,
        grid=(nblk,),
        in_specs=[pl.BlockSpec((be, de), lambda i: (i + blk0, 0)),
                  pl.BlockSpec((d, de), lambda i: (0, 0)),
                  pl.BlockSpec((1, d), lambda i: (0, 0))],
        out_specs=pl.BlockSpec((be, d // 2), lambda i: (i, 0)),
        out_shape=jax.ShapeDtypeStruct((nblk * be, d // 2), jnp.int32),
    )(a, w, b)


def _sc_edge_aggregate(xa, row, col, eaw, eoff):
    n, d = xa.shape
    e = eaw.shape[0]
    nc, ns = 2, 16
    ept = e // ns
    k = 80
    nch = ept // k
    npad = ((n + 2047) // 2048) * 2048
    nh = npad // nc
    rpt = nh // ns
    trows = nh + 8
    crows = nh // d
    chrows = crows + 8

    mesh = plsc.VectorSubcoreMesh(core_axis_name="c", subcore_axis_name="s",
                                  num_cores=nc, num_subcores=ns)

    @functools.partial(
        pl.kernel,
        out_type=(jax.ShapeDtypeStruct((npad, d), _F32),
                  jax.ShapeDtypeStruct((nc * crows, d), _F32)),
        mesh=mesh,
        compiler_params=pltpu.CompilerParams(needs_layout_passes=False),
        scratch_types=[
            pltpu.VMEM((k,), jnp.int32),
            pltpu.VMEM((k,), jnp.int32),
            pltpu.VMEM((k,), jnp.int32),
            pltpu.VMEM((k,), jnp.int32),
            pltpu.VMEM((k,), jnp.int32),
            pltpu.VMEM((k,), jnp.int32),
            pltpu.VMEM((k, d), _F32),
            pltpu.VMEM((k, d), _F32),
            pltpu.VMEM((k, d // 2), jnp.int32),
            pltpu.VMEM((k, d // 2), jnp.int32),
            pltpu.VMEM((k, d), _F32),
            pltpu.VMEM((k, d), _F32),
            pltpu.VMEM((rpt // 8, d), _F32),
            pltpu.VMEM((chrows, d), _F32),
            pltpu.VMEM((chrows,), jnp.int32),
            pltpu.VMEM_SHARED((trows, d), _F32),
            pltpu.VMEM_SHARED((chrows, d), _F32),
            pltpu.SemaphoreType.DMA,
            pltpu.SemaphoreType.DMA,
            pltpu.SemaphoreType.DMA,
            pltpu.SemaphoreType.DMA,
            pltpu.SemaphoreType.DMA,
            pltpu.SemaphoreType.DMA,
            pltpu.SemaphoreType.DMA,
            pltpu.SemaphoreType.DMA,
        ],
    )
    def body(xa_ref, row_ref, col_ref, eaw_ref, out_ref, cnt_ref,
             ri0, ri1, rc0, rc1, sc0, sc1, g0, g1, e0, e1, s0, s1,
             zbuf, hist, crid, acc, cacc,
             semg0, semg1, seme0, seme1, semsc0, semsc1, semi0, semi1):
        cid = lax.axis_index("c")
        sid = lax.axis_index("s")
        lo = cid * nh
        tile_base = sid * ept
        idx_base = eoff + tile_base

        zero16 = jnp.zeros((16,), _F32)
        fones = jnp.ones((16,), _F32)
        lane = lax.iota(jnp.int32, 16)

        ri = (ri0, ri1)
        rc = (rc0, rc1)
        sc = (sc0, sc1)
        gb = (g0, g1)
        eb = (e0, e1)
        sb = (s0, s1)
        semg = (semg0, semg1)
        seme = (seme0, seme1)
        semsc = (semsc0, semsc1)
        semi = (semi0, semi1)

        def zrow(i, carry):
            for j in range(d // 16):
                zbuf[i, pl.ds(j * 16, 16)] = zero16
            return carry
        lax.fori_loop(0, rpt // 8, zrow, 0)

        def hrow(i, carry):
            for j in range(d // 16):
                hist[i, pl.ds(j * 16, 16)] = zero16
            return carry
        lax.fori_loop(0, chrows, hrow, 0)

        for g in range(chrows // 16):
            crid[pl.ds(g * 16, 16)] = lane + (g * 16)

        for b in range(8):
            pltpu.sync_copy(zbuf,
                            acc.at[pl.ds(sid * rpt + b * (rpt // 8), rpt // 8), :])

        @pl.when(sid == 0)
        def _():
            pltpu.sync_copy(zbuf.at[pl.ds(0, chrows), :], cacc)
        plsc.subcore_barrier()

        def issue_idx(p, t):
            base = idx_base + t * k
            pltpu.async_copy(row_ref.at[pl.ds(base, k)], ri[p], semi[p])
            pltpu.async_copy(col_ref.at[pl.ds(base, k)], rc[p], semi[p])

        def wait_idx(p):
            pltpu.make_async_copy(row_ref.at[pl.ds(0, k)], ri[p],
                                  semi[p]).wait()
            pltpu.make_async_copy(col_ref.at[pl.ds(0, k)], rc[p],
                                  semi[p]).wait()

        def issue_fetch(p, t):
            pltpu.async_copy(xa_ref.at[ri[p]], gb[p], semg[p])
            pltpu.async_copy(eaw_ref.at[pl.ds(tile_base + t * k, k), :],
                             eb[p], seme[p])

        issue_idx(0, 0)
        wait_idx(0)
        issue_fetch(0, 0)
        issue_idx(1, 1)

        def phase(t, t2, p):
            q = 1 - p
            tn = jnp.minimum(t + 1, nch - 1)
            wait_idx(q)
            issue_fetch(q, tn)
            pltpu.make_async_copy(xa_ref.at[ri[p]], gb[p], semg[p]).wait()
            pltpu.make_async_copy(
                eaw_ref.at[pl.ds(tile_base, k), :], eb[p], seme[p]).wait()

            @pl.when(t2 >= 1)
            def _():
                pltpu.make_async_copy(sb[p], acc.at[sc[p]], semsc[p]).wait()

            def rowbody(r4, c2):
                for u in range(4):
                    r = r4 * 4 + u
                    for c in range(d // 32):
                        ev = plsc.bitcast(eb[p][r, pl.ds(c * 16, 16)],
                                          jnp.bfloat16)
                        fe, fo = plsc.unpack(
                            ev, format=plsc.PackFormat.INTERLEAVED)
                        sl_lo = pl.ds(c * 16, 16)
                        sl_hi = pl.ds(d // 2 + c * 16, 16)
                        sb[p][r, sl_lo] = jnp.maximum(gb[p][r, sl_lo] + fe,
                                                      0.0)
                        sb[p][r, sl_hi] = jnp.maximum(gb[p][r, sl_hi] + fo,
                                                      0.0)
                return c2
            lax.fori_loop(0, k // 4, rowbody, 0)

            for g in range(k // 16):
                sl = pl.ds(g * 16, 16)
                v = rc[p][sl] - lo
                ok = jnp.logical_and(v >= 0, v < nh)
                sc[p][sl] = jnp.where(ok, v, nh)

            issue_idx(p, jnp.minimum(t + 2, nch - 1))
            pltpu.async_copy(sb[p], acc.at[sc[p]], semsc[p], add=True)

            def cgroup(g, c3):
                colv = sc[p][pl.ds(g * 16, 16)]
                r_i = jnp.right_shift(colv, 7)
                l_i = jnp.bitwise_and(colv, 127)
                for l in range(16):
                    plsc.addupdate_scatter(hist, [r_i, l_i], fones,
                                           mask=lane == l)
                return c3
            lax.fori_loop(0, k // 16, cgroup, jnp.int32(0))

        def outer(t2, carry):
            phase(2 * t2, t2, 0)
            phase(2 * t2 + 1, t2, 1)
            return carry
        lax.fori_loop(0, nch // 2, outer, 0)
        if nch % 2 == 1:
            phase(nch - 1, nch // 2, (nch - 1) % 2)

        pl_last = (nch - 1) % 2
        q_last = 1 - pl_last
        wait_idx(pl_last)
        pltpu.make_async_copy(xa_ref.at[ri[q_last]], gb[q_last],
                              semg[q_last]).wait()
        pltpu.make_async_copy(eaw_ref.at[pl.ds(tile_base, k), :],
                              eb[q_last], seme[q_last]).wait()
        pltpu.make_async_copy(sb[0], acc.at[sc[0]], semsc[0]).wait()
        pltpu.make_async_copy(sb[1], acc.at[sc[1]], semsc[1]).wait()

        pltpu.sync_copy(hist, cacc.at[crid], add=True)
        plsc.subcore_barrier()
        pltpu.sync_copy(acc.at[pl.ds(sid * rpt, rpt), :],
                        out_ref.at[pl.ds(cid * nh + sid * rpt, rpt), :])

        @pl.when(sid == 0)
        def _():
            pltpu.sync_copy(cacc.at[pl.ds(0, crows), :],
                            cnt_ref.at[pl.ds(cid * crows, crows), :])

    return body(xa, row, col, eaw)


def _node_body(p_ref, p2_ref, c4_ref, c42_ref, x_ref, b3_ref, u_ref, w1b_ref,
               b1b_ref, w2a_ref, b2a_ref, w2b_ref, b2b_ref, wv_ref, bv_ref,
               wo_ref, bo_ref, o_ref):
    d = x_ref.shape[1]
    bn = x_ref.shape[0]
    nb = u_ref.shape[0]
    sums = p_ref[...] + p2_ref[...]
    cnt = jnp.reshape(c4_ref[...] + c42_ref[...], (bn, 1))
    mean = sums / jnp.maximum(cnt, 1.0)
    agg = _dot_t(mean, w1b_ref[...]) + jnp.where(cnt > 0.0, 1.0, 0.0) * b1b_ref[...]
    c = _dot_t(u_ref[...], wv_ref[...]) + bv_ref[...]
    c = _dot_t(c, wo_ref[...]) + bo_ref[...]
    w2a = w2a_ref[...]
    cb = _dot_t(c, w2a[:, 2 * d:3 * d])
    bb = jnp.reshape(b3_ref[...], (bn, 1))
    onehot = (bb == lax.broadcasted_iota(jnp.int32, (bn, nb), 1)).astype(_F32)
    pre2 = (_dot_t(x_ref[...], w2a[:, :d])
            + _dot_t(agg, w2a[:, d:2 * d])
            + lax.dot_general(onehot, cb, (((1,), (0,)), ((), ())),
                              preferred_element_type=_F32)
            + b2a_ref[...])
    o_ref[...] = _dot_t(jnp.maximum(pre2, 0.0), w2b_ref[...]) + b2b_ref[...]


def _tc_node_mlp(p, p2, c4, c42, x, batch3, u, w1b, b1b, w2a, b2a, w2b, b2b,
                 wv, bv, wo, bo):
    n, d = x.shape
    nb = u.shape[0]
    bn = 1000
    grid = n // bn
    full = lambda *s: pl.BlockSpec(s, lambda i: tuple(0 for _ in s))
    return pl.pallas_call(
        _node_body,
        grid=(grid,),
        in_specs=[
            pl.BlockSpec((bn, d), lambda i: (i, 0)),
            pl.BlockSpec((bn, d), lambda i: (i, 0)),
            pl.BlockSpec((1, 1, bn), lambda i: (i, 0, 0)),
            pl.BlockSpec((1, 1, bn), lambda i: (i, 0, 0)),
            pl.BlockSpec((bn, d), lambda i: (i, 0)),
            pl.BlockSpec((1, 1, bn), lambda i: (i, 0, 0)),
            full(nb, d),
            full(d, d), full(1, d),
            full(d, 3 * d), full(1, d),
            full(d, d), full(1, d),
            full(d, d), full(1, d),
            full(d, d), full(1, d),
        ],
        out_specs=pl.BlockSpec((bn, d), lambda i: (i, 0)),
        out_shape=jax.ShapeDtypeStruct((n, d), _F32),
    )(p, p2, c4, c42, x, batch3, u, w1b, b1b, w2a, b2a, w2b, b2b, wv, bv,
      wo, bo)


def kernel(x, edge_index, edge_attr, u, batch, W1a, b1a, W1b, b1b, W2a, b2a,
           W2b, b2b, Wq, bq, Wk, bk, Wv, bv, Wo, bo):
    n, d = x.shape
    row = edge_index[0]
    col = edge_index[1]
    e = edge_attr.shape[0]
    be = 2000
    nblk_half = e // (2 * be)
    xa = _tc_xa(x, W1a[:, :d])
    w1ae = W1a[:, d:]
    b1a2 = b1a.reshape(1, d)
    eaw_a = _tc_pack2(edge_attr, w1ae, b1a2, 0, nblk_half)
    sums_a, cnts_a = _sc_edge_aggregate(xa, row, col, eaw_a, 0)
    eaw_b = _tc_pack2(edge_attr, w1ae, b1a2, nblk_half, nblk_half)
    sums_b, cnts_b = _sc_edge_aggregate(xa, row, col, eaw_b, e // 2)
    c4a = cnts_a.reshape(-1)[:n].reshape(n // 1000, 1, 1000)
    c4b = cnts_b.reshape(-1)[:n].reshape(n // 1000, 1, 1000)
    batch3 = batch.reshape(n // 1000, 1, 1000)
    return _tc_node_mlp(sums_a, sums_b, c4a, c4b, x, batch3, u,
                        W1b, b1b.reshape(1, d), W2a, b2a.reshape(1, d),
                        W2b, b2b.reshape(1, d), Wv, bv.reshape(1, d),
                        Wo, bo.reshape(1, d))
